# Initial kernel scaffold; baseline (speedup 1.0000x reference)
import functools

import jax
import jax.numpy as jnp
from jax import lax
from jax.experimental import pallas as pl
from jax.experimental.pallas import tpu as pltpu
from jax.experimental.pallas import tpu_sc as plsc

B = 4096
ED = 32
H_DIM = 128
ITEM_LEN = 50
SLOT_LEN = 20

NW = 32
CHUNK = 128
GROUP = 5
ITEM_CHUNKS = B * ITEM_LEN // (NW * CHUNK)
SLOT_CHUNKS = B * SLOT_LEN // (NW * CHUNK)
ITEM_GROUPS = ITEM_CHUNKS // GROUP
SLOT_GROUPS = SLOT_CHUNKS // GROUP
GROWS = GROUP * CHUNK


def _sc_gather_kernel(items_hbm, slots_hbm, ei_hbm, es_hbm, out_i, out_s,
                      idx_i, idx_s, rows, sem):
    c = lax.axis_index("c")
    s = lax.axis_index("s")
    wid = s * 2 + c

    pltpu.sync_copy(items_hbm.at[wid], idx_i)
    pltpu.sync_copy(slots_hbm.at[wid], idx_s)

    def item_group(g, carry):
        cps = []
        for j in range(GROUP):
            ck = g * GROUP + j
            cps.append(pltpu.async_copy(
                ei_hbm.at[idx_i.at[ck]], rows.at[pl.ds(j * CHUNK, CHUNK)], sem))
        for cp in cps:
            cp.wait()
        base = wid * (ITEM_CHUNKS * CHUNK) + g * GROWS
        pltpu.sync_copy(rows, out_i.at[pl.ds(base, GROWS)])
        return carry

    lax.fori_loop(0, ITEM_GROUPS, item_group, 0)

    def slot_group(g, carry):
        cps = []
        for j in range(GROUP):
            ck = g * GROUP + j
            cps.append(pltpu.async_copy(
                es_hbm.at[idx_s.at[ck]], rows.at[pl.ds(j * CHUNK, CHUNK)], sem))
        for cp in cps:
            cp.wait()
        base = wid * (SLOT_CHUNKS * CHUNK) + g * GROWS
        pltpu.sync_copy(rows, out_s.at[pl.ds(base, GROWS)])
        return carry

    lax.fori_loop(0, SLOT_GROUPS, slot_group, 0)


@jax.jit
def _sc_gather(items_r, slots_r, e_items, e_slots):
    mesh = plsc.VectorSubcoreMesh(core_axis_name="c", subcore_axis_name="s")
    f = functools.partial(
        pl.kernel,
        mesh=mesh,
        out_type=[
            jax.ShapeDtypeStruct((B * ITEM_LEN, ED), jnp.float32),
            jax.ShapeDtypeStruct((B * SLOT_LEN, ED), jnp.float32),
        ],
        scratch_types=[
            pltpu.VMEM((ITEM_CHUNKS, CHUNK), jnp.int32),
            pltpu.VMEM((SLOT_CHUNKS, CHUNK), jnp.int32),
            pltpu.VMEM((GROWS, ED), jnp.float32),
            pltpu.SemaphoreType.DMA,
        ],
    )(_sc_gather_kernel)
    return f(items_r, slots_r, e_items, e_slots)


def _tc_matmul_kernel(xi_ref, xs_ref, wi_ref, bi_ref, ws_ref, bs_ref, out_ref):
    fi = jnp.dot(xi_ref[...], wi_ref[...], preferred_element_type=jnp.float32)
    fi = jnp.maximum(fi + bi_ref[...], 0.0)
    fs = jnp.dot(xs_ref[...], ws_ref[...], preferred_element_type=jnp.float32)
    fs = jnp.maximum(fs + bs_ref[...], 0.0)
    out_ref[...] = jnp.concatenate([fi, fs], axis=1)


BT = 256


@jax.jit
def _tc_matmul(xi, xs, w_items, b_items, w_slots, b_slots):
    grid = (B // BT,)
    return pl.pallas_call(
        _tc_matmul_kernel,
        grid=grid,
        in_specs=[
            pl.BlockSpec((BT, ITEM_LEN * ED), lambda i: (i, 0)),
            pl.BlockSpec((BT, SLOT_LEN * ED), lambda i: (i, 0)),
            pl.BlockSpec((ITEM_LEN * ED, H_DIM), lambda i: (0, 0)),
            pl.BlockSpec((1, H_DIM), lambda i: (0, 0)),
            pl.BlockSpec((SLOT_LEN * ED, H_DIM), lambda i: (0, 0)),
            pl.BlockSpec((1, H_DIM), lambda i: (0, 0)),
        ],
        out_specs=pl.BlockSpec((BT, 2 * H_DIM), lambda i: (i, 0)),
        out_shape=jax.ShapeDtypeStruct((B, 2 * H_DIM), jnp.float32),
    )(xi, xs, w_items, b_items, w_slots, b_slots)


def kernel(items, slots, E_items, E_slots, W_items, b_items, W_slots, b_slots):
    items_r = items.astype(jnp.int32).reshape(NW, ITEM_CHUNKS, CHUNK)
    slots_r = slots.astype(jnp.int32).reshape(NW, SLOT_CHUNKS, CHUNK)
    xi_flat, xs_flat = _sc_gather(items_r, slots_r, E_items, E_slots)
    xi = xi_flat.reshape(B, ITEM_LEN * ED)
    xs = xs_flat.reshape(B, SLOT_LEN * ED)
    return _tc_matmul(xi, xs, W_items, b_items.reshape(1, H_DIM),
                      W_slots, b_slots.reshape(1, H_DIM))


# speedup vs baseline: 9.0342x; 9.0342x over previous
import functools

import jax
import jax.numpy as jnp
from jax import lax
from jax.experimental import pallas as pl
from jax.experimental.pallas import tpu as pltpu
from jax.experimental.pallas import tpu_sc as plsc

B = 4096
ED = 32
H_DIM = 128
ITEM_LEN = 50
SLOT_LEN = 20

NW = 32
CHUNK = 128
GROUP = 5
ITEM_CHUNKS = B * ITEM_LEN // (NW * CHUNK)
SLOT_CHUNKS = B * SLOT_LEN // (NW * CHUNK)
ITEM_GROUPS = ITEM_CHUNKS // GROUP
SLOT_GROUPS = SLOT_CHUNKS // GROUP
GROWS = GROUP * CHUNK


def _sc_gather_kernel(items_hbm, slots_hbm, ei_hbm, es_hbm, out_i, out_s,
                      idx_i, idx_s, rows, sem):
    c = lax.axis_index("c")
    s = lax.axis_index("s")
    wid = s * 2 + c

    pltpu.sync_copy(items_hbm.at[wid], idx_i)
    pltpu.sync_copy(slots_hbm.at[wid], idx_s)

    def item_group(g, carry):
        cps = []
        for j in range(GROUP):
            ck = g * GROUP + j
            cps.append(pltpu.async_copy(
                ei_hbm.at[idx_i.at[ck]], rows.at[pl.ds(j * CHUNK, CHUNK)], sem))
        for cp in cps:
            cp.wait()
        base = wid * (ITEM_CHUNKS * CHUNK) + g * GROWS
        pltpu.sync_copy(rows, out_i.at[pl.ds(base, GROWS)])
        return carry

    lax.fori_loop(0, ITEM_GROUPS, item_group, 0)

    def slot_group(g, carry):
        cps = []
        for j in range(GROUP):
            ck = g * GROUP + j
            cps.append(pltpu.async_copy(
                es_hbm.at[idx_s.at[ck]], rows.at[pl.ds(j * CHUNK, CHUNK)], sem))
        for cp in cps:
            cp.wait()
        base = wid * (SLOT_CHUNKS * CHUNK) + g * GROWS
        pltpu.sync_copy(rows, out_s.at[pl.ds(base, GROWS)])
        return carry

    lax.fori_loop(0, SLOT_GROUPS, slot_group, 0)


@jax.jit
def _sc_gather(items_r, slots_r, e_items, e_slots):
    mesh = plsc.VectorSubcoreMesh(core_axis_name="c", subcore_axis_name="s")
    f = functools.partial(
        pl.kernel,
        mesh=mesh,
        out_type=[
            jax.ShapeDtypeStruct((B * ITEM_LEN, ED), jnp.float32),
            jax.ShapeDtypeStruct((B * SLOT_LEN, ED), jnp.float32),
        ],
        scratch_types=[
            pltpu.VMEM((ITEM_CHUNKS, CHUNK), jnp.int32),
            pltpu.VMEM((SLOT_CHUNKS, CHUNK), jnp.int32),
            pltpu.VMEM((GROWS, ED), jnp.float32),
            pltpu.SemaphoreType.DMA,
        ],
        compiler_params=pltpu.CompilerParams(use_tc_tiling_on_sc=False),
    )(_sc_gather_kernel)
    return f(items_r, slots_r, e_items, e_slots)


def _tc_matmul_kernel(xi_ref, xs_ref, wi_ref, bi_ref, ws_ref, bs_ref, out_ref):
    fi = jnp.dot(xi_ref[...], wi_ref[...], preferred_element_type=jnp.float32)
    fi = jnp.maximum(fi + bi_ref[...], 0.0)
    fs = jnp.dot(xs_ref[...], ws_ref[...], preferred_element_type=jnp.float32)
    fs = jnp.maximum(fs + bs_ref[...], 0.0)
    out_ref[...] = jnp.concatenate([fi, fs], axis=1)


BT = 256


@jax.jit
def _tc_matmul(xi, xs, w_items, b_items, w_slots, b_slots):
    grid = (B // BT,)
    return pl.pallas_call(
        _tc_matmul_kernel,
        grid=grid,
        in_specs=[
            pl.BlockSpec((BT, ITEM_LEN * ED), lambda i: (i, 0)),
            pl.BlockSpec((BT, SLOT_LEN * ED), lambda i: (i, 0)),
            pl.BlockSpec((ITEM_LEN * ED, H_DIM), lambda i: (0, 0)),
            pl.BlockSpec((1, H_DIM), lambda i: (0, 0)),
            pl.BlockSpec((SLOT_LEN * ED, H_DIM), lambda i: (0, 0)),
            pl.BlockSpec((1, H_DIM), lambda i: (0, 0)),
        ],
        out_specs=pl.BlockSpec((BT, 2 * H_DIM), lambda i: (i, 0)),
        out_shape=jax.ShapeDtypeStruct((B, 2 * H_DIM), jnp.float32),
    )(xi, xs, w_items, b_items, w_slots, b_slots)


def kernel(items, slots, E_items, E_slots, W_items, b_items, W_slots, b_slots):
    items_r = items.astype(jnp.int32).reshape(NW, ITEM_CHUNKS, CHUNK)
    slots_r = slots.astype(jnp.int32).reshape(NW, SLOT_CHUNKS, CHUNK)
    xi_flat, xs_flat = _sc_gather(items_r, slots_r, E_items, E_slots)
    xi = xi_flat.reshape(B, ITEM_LEN * ED)
    xs = xs_flat.reshape(B, SLOT_LEN * ED)
    return _tc_matmul(xi, xs, W_items, b_items.reshape(1, H_DIM),
                      W_slots, b_slots.reshape(1, H_DIM))


# speedup vs baseline: 27.0520x; 2.9944x over previous
import functools

import jax
import jax.numpy as jnp
from jax import lax
from jax.experimental import pallas as pl
from jax.experimental.pallas import tpu as pltpu
from jax.experimental.pallas import tpu_sc as plsc

B = 4096
ED = 32
H_DIM = 128
ITEM_LEN = 50
SLOT_LEN = 20

NW = 32
CHUNK = 128
GROUP = 10
ITEM_CHUNKS = B * ITEM_LEN // (NW * CHUNK)
SLOT_CHUNKS = B * SLOT_LEN // (NW * CHUNK)
ITEM_GROUPS = ITEM_CHUNKS // GROUP
SLOT_GROUPS = SLOT_CHUNKS // GROUP
GROWS = GROUP * CHUNK


def _sc_gather_kernel(items_hbm, slots_hbm, ei_hbm, es_hbm, out_i, out_s,
                      idx_i, idx_s, rows_a, rows_b, gsem_a, gsem_b,
                      wsem_a, wsem_b):
    c = lax.axis_index("c")
    s = lax.axis_index("s")
    wid = s * 2 + c

    pltpu.sync_copy(items_hbm.at[wid], idx_i)
    pltpu.sync_copy(slots_hbm.at[wid], idx_s)

    bufs = (rows_a, rows_b)
    gsems = (gsem_a, gsem_b)
    wsems = (wsem_a, wsem_b)

    def run(idx_v, tab_hbm, out_hbm, n_chunks, first):
        n_groups = n_chunks // GROUP
        stride = n_chunks * CHUNK

        def dummy_wait(sem):
            pltpu.make_async_copy(out_hbm.at[pl.ds(0, GROWS)],
                                  bufs[0], sem).wait()

        def fire(g, b):
            for j in range(GROUP):
                ck = g * GROUP + j
                pltpu.async_copy(tab_hbm.at[idx_v.at[ck]],
                                 bufs[b].at[pl.ds(j * CHUNK, CHUNK)],
                                 gsems[b])

        def step(g, carry):
            for b in (0, 1):
                @pl.when(jnp.logical_and(g % 2 == b, g < n_groups))
                def _(b=b):
                    @pl.when((g >= 2) if first else (g >= 0))
                    def _():
                        dummy_wait(wsems[b])

                    fire(g, b)

            for pb in (0, 1):
                @pl.when(jnp.logical_and((g - 1) % 2 == pb, g >= 1))
                def _(pb=pb):
                    dummy_wait(gsems[pb])
                    base = wid * stride + (g - 1) * GROWS
                    pltpu.async_copy(bufs[pb],
                                     out_hbm.at[pl.ds(base, GROWS)],
                                     wsems[pb])

            return carry

        lax.fori_loop(0, n_groups + 1, step, 0)

    run(idx_i, ei_hbm, out_i, ITEM_CHUNKS, True)
    run(idx_s, es_hbm, out_s, SLOT_CHUNKS, False)
    pltpu.make_async_copy(out_s.at[pl.ds(0, GROWS)], rows_a, wsem_a).wait()
    pltpu.make_async_copy(out_s.at[pl.ds(0, GROWS)], rows_b, wsem_b).wait()


def _sc_gather(items_r, slots_r, e_items, e_slots):
    mesh = plsc.VectorSubcoreMesh(core_axis_name="c", subcore_axis_name="s")
    f = functools.partial(
        pl.kernel,
        mesh=mesh,
        out_type=[
            jax.ShapeDtypeStruct((B * ITEM_LEN, ED), jnp.float32),
            jax.ShapeDtypeStruct((B * SLOT_LEN, ED), jnp.float32),
        ],
        scratch_types=[
            pltpu.VMEM((ITEM_CHUNKS, CHUNK), jnp.int32),
            pltpu.VMEM((SLOT_CHUNKS, CHUNK), jnp.int32),
            pltpu.VMEM((GROWS, ED), jnp.float32),
            pltpu.VMEM((GROWS, ED), jnp.float32),
            pltpu.SemaphoreType.DMA,
            pltpu.SemaphoreType.DMA,
            pltpu.SemaphoreType.DMA,
            pltpu.SemaphoreType.DMA,
        ],
        compiler_params=pltpu.CompilerParams(use_tc_tiling_on_sc=False),
    )(_sc_gather_kernel)
    return f(items_r, slots_r, e_items, e_slots)


DQ = 16384
DQ_BITS = 14


def _detile_kernel(et_ref, out_ref):
    x = et_ref[...]
    z = jnp.concatenate([x[:, DQ * k:DQ * (k + 1)] for k in range(4)], axis=0)
    out_ref[...] = z.T


@jax.jit
def _detile(et):
    v = et.shape[1]
    grid = (pl.cdiv(v, 4 * DQ),)
    return pl.pallas_call(
        _detile_kernel,
        grid=grid,
        in_specs=[pl.BlockSpec((ED, 4 * DQ), lambda i: (0, i))],
        out_specs=pl.BlockSpec((DQ, 128), lambda i: (i, 0)),
        out_shape=jax.ShapeDtypeStruct((grid[0] * DQ, 128), jnp.float32),
        compiler_params=pltpu.CompilerParams(
            fuse_transposed_lhs_in_matmul=True),
    )(et)


def _permute_idx(idx):
    m = idx & (4 * DQ - 1)
    return (idx - m) + ((m & (DQ - 1)) << 2) + (m >> DQ_BITS)


def _tc_matmul_kernel(xi_ref, xs_ref, wi_ref, bi_ref, ws_ref, bs_ref, out_ref):
    fi = jnp.dot(xi_ref[...], wi_ref[...], preferred_element_type=jnp.float32)
    fi = jnp.maximum(fi + bi_ref[...], 0.0)
    fs = jnp.dot(xs_ref[...], ws_ref[...], preferred_element_type=jnp.float32)
    fs = jnp.maximum(fs + bs_ref[...], 0.0)
    out_ref[...] = jnp.concatenate([fi, fs], axis=1)


BT = 256


@jax.jit
def _tc_matmul(xi, xs, w_items, b_items, w_slots, b_slots):
    grid = (xi.shape[0] // BT,)
    return pl.pallas_call(
        _tc_matmul_kernel,
        grid=grid,
        in_specs=[
            pl.BlockSpec((BT, ITEM_LEN * ED), lambda i: (i, 0)),
            pl.BlockSpec((BT, SLOT_LEN * ED), lambda i: (i, 0)),
            pl.BlockSpec((ITEM_LEN * ED, H_DIM), lambda i: (0, 0)),
            pl.BlockSpec((1, H_DIM), lambda i: (0, 0)),
            pl.BlockSpec((SLOT_LEN * ED, H_DIM), lambda i: (0, 0)),
            pl.BlockSpec((1, H_DIM), lambda i: (0, 0)),
        ],
        out_specs=pl.BlockSpec((BT, 2 * H_DIM), lambda i: (i, 0)),
        out_shape=jax.ShapeDtypeStruct((xi.shape[0], 2 * H_DIM), jnp.float32),
    )(xi, xs, w_items, b_items, w_slots, b_slots)


def kernel(items, slots, E_items, E_slots, W_items, b_items, W_slots, b_slots):
    items_r = _permute_idx(items.astype(jnp.int32)).reshape(
        NW, ITEM_CHUNKS, CHUNK)
    slots_r = _permute_idx(slots.astype(jnp.int32)).reshape(
        NW, SLOT_CHUNKS, CHUNK)
    ei_lin = _detile(E_items.T).reshape(-1, ED)
    es_lin = _detile(E_slots.T).reshape(-1, ED)
    xi_flat, xs_flat = _sc_gather(items_r, slots_r, ei_lin, es_lin)
    xi = xi_flat.reshape(B, ITEM_LEN * ED)
    xs = xs_flat.reshape(B, SLOT_LEN * ED)
    return _tc_matmul(xi, xs, W_items, b_items.reshape(1, H_DIM),
                      W_slots, b_slots.reshape(1, H_DIM))


# speedup vs baseline: 27.0584x; 1.0002x over previous
import functools

import jax
import jax.numpy as jnp
from jax import lax
from jax.experimental import pallas as pl
from jax.experimental.pallas import tpu as pltpu
from jax.experimental.pallas import tpu_sc as plsc

B = 4096
ED = 32
H_DIM = 128
ITEM_LEN = 50
SLOT_LEN = 20

NW = 32
CHUNK = 128
GROUP = 10
ITEM_CHUNKS = B * ITEM_LEN // (NW * CHUNK)
SLOT_CHUNKS = B * SLOT_LEN // (NW * CHUNK)
ITEM_GROUPS = ITEM_CHUNKS // GROUP
SLOT_GROUPS = SLOT_CHUNKS // GROUP
GROWS = GROUP * CHUNK


def _sc_gather_kernel(items_hbm, slots_hbm, ei_hbm, es_hbm, out_i, out_s,
                      idx_i, idx_s, rows_a, rows_b, gsem_a, gsem_b,
                      wsem_a, wsem_b):
    c = lax.axis_index("c")
    s = lax.axis_index("s")
    wid = s * 2 + c

    pltpu.sync_copy(items_hbm.at[wid], idx_i)
    pltpu.sync_copy(slots_hbm.at[wid], idx_s)

    bufs = (rows_a, rows_b)
    gsems = (gsem_a, gsem_b)
    wsems = (wsem_a, wsem_b)

    def run(idx_v, tab_hbm, out_hbm, n_chunks, first):
        n_groups = n_chunks // GROUP
        stride = n_chunks * CHUNK

        def dummy_wait(sem):
            pltpu.make_async_copy(out_hbm.at[pl.ds(0, GROWS)],
                                  bufs[0], sem).wait()

        def fire(g, b):
            for j in range(GROUP):
                ck = g * GROUP + j
                pltpu.async_copy(tab_hbm.at[idx_v.at[ck]],
                                 bufs[b].at[pl.ds(j * CHUNK, CHUNK)],
                                 gsems[b])

        def step(g, carry):
            for b in (0, 1):
                @pl.when(jnp.logical_and(g % 2 == b, g < n_groups))
                def _(b=b):
                    @pl.when((g >= 2) if first else (g >= 0))
                    def _():
                        dummy_wait(wsems[b])

                    fire(g, b)

            for pb in (0, 1):
                @pl.when(jnp.logical_and((g - 1) % 2 == pb, g >= 1))
                def _(pb=pb):
                    dummy_wait(gsems[pb])
                    base = wid * stride + (g - 1) * GROWS
                    pltpu.async_copy(bufs[pb],
                                     out_hbm.at[pl.ds(base, GROWS)],
                                     wsems[pb])

            return carry

        lax.fori_loop(0, n_groups + 1, step, 0)

    run(idx_i, ei_hbm, out_i, ITEM_CHUNKS, True)
    run(idx_s, es_hbm, out_s, SLOT_CHUNKS, False)
    pltpu.make_async_copy(out_s.at[pl.ds(0, GROWS)], rows_a, wsem_a).wait()
    pltpu.make_async_copy(out_s.at[pl.ds(0, GROWS)], rows_b, wsem_b).wait()


def _sc_gather(items_r, slots_r, e_items, e_slots):
    mesh = plsc.VectorSubcoreMesh(core_axis_name="c", subcore_axis_name="s")
    f = functools.partial(
        pl.kernel,
        mesh=mesh,
        out_type=[
            jax.ShapeDtypeStruct((B * ITEM_LEN, ED), jnp.float32),
            jax.ShapeDtypeStruct((B * SLOT_LEN, ED), jnp.float32),
        ],
        scratch_types=[
            pltpu.VMEM((ITEM_CHUNKS, CHUNK), jnp.int32),
            pltpu.VMEM((SLOT_CHUNKS, CHUNK), jnp.int32),
            pltpu.VMEM((GROWS, ED), jnp.float32),
            pltpu.VMEM((GROWS, ED), jnp.float32),
            pltpu.SemaphoreType.DMA,
            pltpu.SemaphoreType.DMA,
            pltpu.SemaphoreType.DMA,
            pltpu.SemaphoreType.DMA,
        ],
        compiler_params=pltpu.CompilerParams(use_tc_tiling_on_sc=False),
    )(_sc_gather_kernel)
    return f(items_r, slots_r, e_items, e_slots)


DQ = 16384
DQ_BITS = 14


def _detile_kernel(et_ref, out_ref):
    x = et_ref[...]
    z = jnp.concatenate([x[:, DQ * k:DQ * (k + 1)] for k in range(4)], axis=0)
    out_ref[...] = z.T


@jax.jit
def _detile(et):
    v = et.shape[1]
    grid = (pl.cdiv(v, 4 * DQ),)
    return pl.pallas_call(
        _detile_kernel,
        grid=grid,
        in_specs=[pl.BlockSpec((ED, 4 * DQ), lambda i: (0, i))],
        out_specs=pl.BlockSpec((DQ, 128), lambda i: (i, 0)),
        out_shape=jax.ShapeDtypeStruct((grid[0] * DQ, 128), jnp.float32),
    )(et)


def _permute_idx(idx):
    m = idx & (4 * DQ - 1)
    return (idx - m) + ((m & (DQ - 1)) << 2) + (m >> DQ_BITS)


def _tc_matmul_kernel(xi_ref, xs_ref, wi_ref, bi_ref, ws_ref, bs_ref, out_ref):
    fi = jnp.dot(xi_ref[...], wi_ref[...], preferred_element_type=jnp.float32)
    fi = jnp.maximum(fi + bi_ref[...], 0.0)
    fs = jnp.dot(xs_ref[...], ws_ref[...], preferred_element_type=jnp.float32)
    fs = jnp.maximum(fs + bs_ref[...], 0.0)
    out_ref[...] = jnp.concatenate([fi, fs], axis=1)


BT = 256


@jax.jit
def _tc_matmul(xi, xs, w_items, b_items, w_slots, b_slots):
    grid = (xi.shape[0] // BT,)
    return pl.pallas_call(
        _tc_matmul_kernel,
        grid=grid,
        in_specs=[
            pl.BlockSpec((BT, ITEM_LEN * ED), lambda i: (i, 0)),
            pl.BlockSpec((BT, SLOT_LEN * ED), lambda i: (i, 0)),
            pl.BlockSpec((ITEM_LEN * ED, H_DIM), lambda i: (0, 0)),
            pl.BlockSpec((1, H_DIM), lambda i: (0, 0)),
            pl.BlockSpec((SLOT_LEN * ED, H_DIM), lambda i: (0, 0)),
            pl.BlockSpec((1, H_DIM), lambda i: (0, 0)),
        ],
        out_specs=pl.BlockSpec((BT, 2 * H_DIM), lambda i: (i, 0)),
        out_shape=jax.ShapeDtypeStruct((xi.shape[0], 2 * H_DIM), jnp.float32),
    )(xi, xs, w_items, b_items, w_slots, b_slots)


def kernel(items, slots, E_items, E_slots, W_items, b_items, W_slots, b_slots):
    items_r = _permute_idx(items.astype(jnp.int32)).reshape(
        NW, ITEM_CHUNKS, CHUNK)
    slots_r = _permute_idx(slots.astype(jnp.int32)).reshape(
        NW, SLOT_CHUNKS, CHUNK)
    ei_lin = _detile(E_items.T).reshape(-1, ED)
    es_lin = _detile(E_slots.T).reshape(-1, ED)
    xi_flat, xs_flat = _sc_gather(items_r, slots_r, ei_lin, es_lin)
    xi = xi_flat.reshape(B, ITEM_LEN * ED)
    xs = xs_flat.reshape(B, SLOT_LEN * ED)
    return _tc_matmul(xi, xs, W_items, b_items.reshape(1, H_DIM),
                      W_slots, b_slots.reshape(1, H_DIM))
